# baseline (speedup 1.0000x reference)
import functools

import jax
import jax.numpy as jnp
from jax import lax
from jax.experimental import pallas as pl
from jax.experimental.pallas import tpu as pltpu
from jax.experimental.pallas import tpu_sc as plsc

NBINS = 16384
SHIFT = 32 - 14
NCORES = 2
NSUB = 16
NTILES = NCORES * NSUB
LANES = 16


def _ce_body(labels_ref, weights_ref, logits_ref, loss_ref):
    lg = logits_ref[0, 0]
    lab = labels_ref[0]
    c_dim = lg.shape[0]
    m = jnp.max(lg, axis=0)
    se = jnp.sum(jnp.exp(lg - m[None]), axis=0)
    lse = jnp.log(se) + m
    picked = jnp.zeros_like(lse)
    wp = jnp.zeros_like(lse)
    for c in range(c_dim):
        sel = lab == c
        picked = jnp.where(sel, lg[c], picked)
        wp = jnp.where(sel, weights_ref[0, c], wp)
    loss_ref[0] = (lse - picked) * wp


def _ce_losses_scale(logits, labels3, weights2, scale, b0, nb):
    s, b, c, h, w = logits.shape
    bh = 256
    grid = (nb, h // bh)
    out = pl.pallas_call(
        _ce_body,
        grid=grid,
        in_specs=[
            pl.BlockSpec((1, bh, w), lambda j, r, B=b0: (B + j, r, 0)),
            pl.BlockSpec(memory_space=pltpu.SMEM),
            pl.BlockSpec((1, 1, c, bh, w),
                         lambda j, r, S=scale, B=b0: (S, B + j, 0, r, 0)),
        ],
        out_specs=pl.BlockSpec((1, bh, w), lambda j, r: (j, r, 0)),
        out_shape=jax.ShapeDtypeStruct((nb, h, w), jnp.float32),
    )(labels3, weights2, logits)
    return out.reshape(nb * h, w)


def _hist_body(loss_hbm, cnt_hbm, sum_hbm, buf, hcnt, hsum, hcnt2, hsum2):
    nr, nw = loss_hbm.shape
    rows = nr // NTILES
    wid = lax.axis_index("s") * NCORES + lax.axis_index("c")

    zeros16 = jnp.zeros((LANES,), jnp.float32)
    ones16 = jnp.ones((LANES,), jnp.float32)

    def zero_body(i, _):
        sl = pl.ds(i * LANES, LANES)
        hcnt[sl] = zeros16
        hsum[sl] = zeros16
        hcnt2[sl] = zeros16
        hsum2[sl] = zeros16
        return 0

    lax.fori_loop(0, NBINS // LANES, zero_body, 0)

    pltpu.sync_copy(loss_hbm.at[pl.ds(wid * rows, rows)], buf)

    @plsc.parallel_loop(0, rows, unroll=2)
    def scat_row(r):
        for c in range(nw // LANES):
            vals = buf[r, pl.ds(c * LANES, LANES)]
            bits = lax.bitcast_convert_type(vals, jnp.int32)
            idx = lax.shift_right_logical(bits, SHIFT)
            hc = hcnt if c % 2 == 0 else hcnt2
            hs = hsum if c % 2 == 0 else hsum2
            plsc.addupdate_scatter(hc, [idx], ones16)
            plsc.addupdate_scatter(hs, [idx], vals)

    @plsc.parallel_loop(0, NBINS // LANES, unroll=2)
    def merge(i):
        sl = pl.ds(i * LANES, LANES)
        hcnt[sl] = hcnt[sl] + hcnt2[sl]
        hsum[sl] = hsum[sl] + hsum2[sl]

    pltpu.sync_copy(hcnt, cnt_hbm.at[wid])
    pltpu.sync_copy(hsum, sum_hbm.at[wid])


def _sc_histogram(losses):
    nr, nw = losses.shape
    out_t = jax.ShapeDtypeStruct((NTILES, NBINS), jnp.float32)
    call = pl.kernel(
        _hist_body,
        out_type=[out_t, out_t],
        mesh=plsc.VectorSubcoreMesh(core_axis_name="c", subcore_axis_name="s"),
        compiler_params=pltpu.CompilerParams(needs_layout_passes=False),
        scratch_types=[
            pltpu.VMEM((nr // NTILES, nw), jnp.float32),
            pltpu.VMEM((NBINS,), jnp.float32),
            pltpu.VMEM((NBINS,), jnp.float32),
            pltpu.VMEM((NBINS,), jnp.float32),
            pltpu.VMEM((NBINS,), jnp.float32),
        ],
    )
    return call(losses)


def _cumsum_lanes(x):
    n = x.shape[1]
    sh = 1
    while sh < n:
        pad = jnp.zeros((x.shape[0], sh), x.dtype)
        x = x + jnp.concatenate([pad, x[:, :-sh]], axis=1)
        sh *= 2
    return x


def _finalize_body(k_top, groups, *refs):
    out_ref = refs[-1]
    refs = refs[:-1]
    cnt_rows, sms_rows = [], []
    i = 0
    for g in groups:
        c_acc, s_acc = None, None
        for _ in range(g):
            c = jnp.sum(refs[i][...], axis=0, keepdims=True)
            s2 = jnp.sum(refs[i + 1][...], axis=0, keepdims=True)
            c_acc = c if c_acc is None else c_acc + c
            s_acc = s2 if s_acc is None else s_acc + s2
            i += 2
        cnt_rows.append(c_acc)
        sms_rows.append(s_acc)
    cnt = jnp.concatenate(cnt_rows, axis=0)
    sms = jnp.concatenate(sms_rows, axis=0)
    cum_c = _cumsum_lanes(cnt)
    cum_s = _cumsum_lanes(sms)
    tot_c = cum_c[:, NBINS - 1 : NBINS]
    tot_s = cum_s[:, NBINS - 1 : NBINS]
    incl_c = tot_c - cum_c + cnt
    incl_s = tot_s - cum_s + sms
    kf = jnp.float32(k_top)
    mask = (incl_c >= kf).astype(jnp.float32)
    bf = jnp.sum(mask, axis=1, keepdims=True) - 1.0
    bi = bf.astype(jnp.int32)
    iota = lax.broadcasted_iota(jnp.int32, cnt.shape, 1)
    oh = iota == bi
    zero = jnp.zeros_like(cnt)
    c_b = jnp.sum(jnp.where(oh, cnt, zero), axis=1, keepdims=True)
    s_b = jnp.sum(jnp.where(oh, sms, zero), axis=1, keepdims=True)
    i_c = jnp.sum(jnp.where(oh, incl_c, zero), axis=1, keepdims=True)
    i_s = jnp.sum(jnp.where(oh, incl_s, zero), axis=1, keepdims=True)
    cab = i_c - c_b
    sab = i_s - s_b
    lo = lax.bitcast_convert_type(bi << SHIFT, jnp.float32)
    hi = lax.bitcast_convert_type((bi + 1) << SHIFT, jnp.float32)
    width = hi - lo
    j = kf - cab
    mu = s_b / c_b
    topj = j * mu + j * (1.0 - j / c_b) * width * 0.5
    mean_s = (sab + topj) / kf
    out_ref[0, 0] = jnp.sum(mean_s) / mean_s.shape[0]


def _finalize(hists, k_top, groups):
    out = pl.pallas_call(
        functools.partial(_finalize_body, k_top, tuple(groups)),
        out_specs=pl.BlockSpec(memory_space=pltpu.SMEM),
        out_shape=jax.ShapeDtypeStruct((1, 1), jnp.float32),
    )(*hists)
    return out[0, 0]


def kernel(logits, labels, weights):
    nscales, b, c, h, w = logits.shape
    labels3 = labels.reshape(b, h, w)
    weights2 = weights.reshape(1, c)
    hists = []
    groups = []
    for s in range(nscales):
        chunks = [(0, b)]
        groups.append(len(chunks))
        for b0, nb in chunks:
            losses = _ce_losses_scale(logits, labels3, weights2, s, b0, nb)
            hists.extend(_sc_histogram(losses))
    k_top = (b * h * w) * 50 // 100
    return _finalize(hists, k_top, groups)


# speedup vs baseline: 1.0164x; 1.0164x over previous
import functools

import jax
import jax.numpy as jnp
from jax import lax
from jax.experimental import pallas as pl
from jax.experimental.pallas import tpu as pltpu
from jax.experimental.pallas import tpu_sc as plsc

NBINS = 16384
SHIFT = 32 - 14
NCORES = 2
NSUB = 16
NTILES = NCORES * NSUB
LANES = 16


def _ce_body(labels_ref, weights_ref, logits_ref, loss_ref):
    lg = logits_ref[0, 0]
    lab = labels_ref[0]
    c_dim = lg.shape[0]
    m = jnp.max(lg, axis=0)
    se = jnp.sum(jnp.exp(lg - m[None]), axis=0)
    lse = jnp.log(se) + m
    picked = jnp.zeros_like(lse)
    wp = jnp.zeros_like(lse)
    for c in range(c_dim):
        sel = lab == c
        picked = jnp.where(sel, lg[c], picked)
        wp = jnp.where(sel, weights_ref[0, c], wp)
    loss_ref[0] = (lse - picked) * wp


def _ce_losses_scale(logits, labels3, weights2, scale, b0, nb):
    s, b, c, h, w = logits.shape
    bh = 256
    grid = (nb, h // bh)
    out = pl.pallas_call(
        _ce_body,
        grid=grid,
        in_specs=[
            pl.BlockSpec((1, bh, w), lambda j, r, B=b0: (B + j, r, 0)),
            pl.BlockSpec(memory_space=pltpu.SMEM),
            pl.BlockSpec((1, 1, c, bh, w),
                         lambda j, r, S=scale, B=b0: (S, B + j, 0, r, 0)),
        ],
        out_specs=pl.BlockSpec((1, bh, w), lambda j, r: (j, r, 0)),
        out_shape=jax.ShapeDtypeStruct((nb, h, w), jnp.float32),
    )(labels3, weights2, logits)
    return out.reshape(nb * h, w)


def _hist_body(loss_hbm, cnt_hbm, sum_hbm, buf, hcnt, hsum):
    nr, nw = loss_hbm.shape
    rows = nr // NTILES
    wid = lax.axis_index("s") * NCORES + lax.axis_index("c")

    zeros16 = jnp.zeros((LANES,), jnp.float32)
    ones16 = jnp.ones((LANES,), jnp.float32)

    def zero_body(i, _):
        hcnt[pl.ds(i * LANES, LANES)] = zeros16
        hsum[pl.ds(i * LANES, LANES)] = zeros16
        return 0

    lax.fori_loop(0, NBINS // LANES, zero_body, 0)

    pltpu.sync_copy(loss_hbm.at[pl.ds(wid * rows, rows)], buf)

    @plsc.parallel_loop(0, rows, unroll=2)
    def scat_row(r):
        for c in range(nw // LANES):
            vals = buf[r, pl.ds(c * LANES, LANES)]
            bits = lax.bitcast_convert_type(vals, jnp.int32)
            idx = lax.shift_right_logical(bits, SHIFT)
            plsc.addupdate_scatter(hcnt, [idx], ones16)
            plsc.addupdate_scatter(hsum, [idx], vals)

    pltpu.sync_copy(hcnt, cnt_hbm.at[wid])
    pltpu.sync_copy(hsum, sum_hbm.at[wid])


def _sc_histogram(losses):
    nr, nw = losses.shape
    out_t = jax.ShapeDtypeStruct((NTILES, NBINS), jnp.float32)
    call = pl.kernel(
        _hist_body,
        out_type=[out_t, out_t],
        mesh=plsc.VectorSubcoreMesh(core_axis_name="c", subcore_axis_name="s"),
        compiler_params=pltpu.CompilerParams(needs_layout_passes=False),
        scratch_types=[
            pltpu.VMEM((nr // NTILES, nw), jnp.float32),
            pltpu.VMEM((NBINS,), jnp.float32),
            pltpu.VMEM((NBINS,), jnp.float32),
        ],
    )
    return call(losses)


def _cumsum_lanes(x):
    n = x.shape[1]
    sh = 1
    while sh < n:
        pad = jnp.zeros((x.shape[0], sh), x.dtype)
        x = x + jnp.concatenate([pad, x[:, :-sh]], axis=1)
        sh *= 2
    return x


def _finalize_body(k_top, groups, *refs):
    out_ref = refs[-1]
    refs = refs[:-1]
    cnt_rows, sms_rows = [], []
    i = 0
    for g in groups:
        c_acc, s_acc = None, None
        for _ in range(g):
            c = jnp.sum(refs[i][...], axis=0, keepdims=True)
            s2 = jnp.sum(refs[i + 1][...], axis=0, keepdims=True)
            c_acc = c if c_acc is None else c_acc + c
            s_acc = s2 if s_acc is None else s_acc + s2
            i += 2
        cnt_rows.append(c_acc)
        sms_rows.append(s_acc)
    cnt = jnp.concatenate(cnt_rows, axis=0)
    sms = jnp.concatenate(sms_rows, axis=0)
    cum_c = _cumsum_lanes(cnt)
    cum_s = _cumsum_lanes(sms)
    tot_c = cum_c[:, NBINS - 1 : NBINS]
    tot_s = cum_s[:, NBINS - 1 : NBINS]
    incl_c = tot_c - cum_c + cnt
    incl_s = tot_s - cum_s + sms
    kf = jnp.float32(k_top)
    mask = (incl_c >= kf).astype(jnp.float32)
    bf = jnp.sum(mask, axis=1, keepdims=True) - 1.0
    bi = bf.astype(jnp.int32)
    iota = lax.broadcasted_iota(jnp.int32, cnt.shape, 1)
    oh = iota == bi
    zero = jnp.zeros_like(cnt)
    c_b = jnp.sum(jnp.where(oh, cnt, zero), axis=1, keepdims=True)
    s_b = jnp.sum(jnp.where(oh, sms, zero), axis=1, keepdims=True)
    i_c = jnp.sum(jnp.where(oh, incl_c, zero), axis=1, keepdims=True)
    i_s = jnp.sum(jnp.where(oh, incl_s, zero), axis=1, keepdims=True)
    cab = i_c - c_b
    sab = i_s - s_b
    lo = lax.bitcast_convert_type(bi << SHIFT, jnp.float32)
    hi = lax.bitcast_convert_type((bi + 1) << SHIFT, jnp.float32)
    width = hi - lo
    j = kf - cab
    mu = s_b / c_b
    topj = j * mu + j * (1.0 - j / c_b) * width * 0.5
    mean_s = (sab + topj) / kf
    out_ref[0, 0] = jnp.sum(mean_s) / mean_s.shape[0]


def _finalize(hists, k_top, groups):
    out = pl.pallas_call(
        functools.partial(_finalize_body, k_top, tuple(groups)),
        out_specs=pl.BlockSpec(memory_space=pltpu.SMEM),
        out_shape=jax.ShapeDtypeStruct((1, 1), jnp.float32),
    )(*hists)
    return out[0, 0]


def kernel(logits, labels, weights):
    nscales, b, c, h, w = logits.shape
    labels3 = labels.reshape(b, h, w)
    weights2 = weights.reshape(1, c)
    hists = []
    groups = []
    for s in range(nscales):
        chunks = [(0, b)]
        groups.append(len(chunks))
        for b0, nb in chunks:
            losses = _ce_losses_scale(logits, labels3, weights2, s, b0, nb)
            hists.extend(_sc_histogram(losses))
    k_top = (b * h * w) * 50 // 100
    return _finalize(hists, k_top, groups)
